# baseline (speedup 1.0000x reference)
import functools

import jax
import jax.numpy as jnp
from jax import lax
from jax.experimental import pallas as pl
from jax.experimental.pallas import tpu as pltpu
from jax.experimental.pallas import tpu_sc as plsc

B = 4096
D = 64
HIST = 50
HP = 64
NC = 2
NS = 16
NW = NC * NS
BPW = B // NW
RPC = 2
NCHUNK = BPW // RPC
IDX_PER_CHUNK = RPC * HP
LN = 16

_mesh = plsc.VectorSubcoreMesh(
    core_axis_name="c", subcore_axis_name="s", num_cores=NC, num_subcores=NS
)


@functools.partial(
    pl.kernel,
    out_type=jax.ShapeDtypeStruct((B, D), jnp.float32),
    mesh=_mesh,
    scratch_types=[
        pltpu.VMEM((NCHUNK, IDX_PER_CHUNK), jnp.int32),
        pltpu.VMEM((IDX_PER_CHUNK, D), jnp.float32),
        pltpu.VMEM((IDX_PER_CHUNK, D), jnp.float32),
        pltpu.VMEM((BPW, D), jnp.float32),
        pltpu.SemaphoreType.DMA,
        pltpu.SemaphoreType.DMA,
    ],
    compiler_params=pltpu.CompilerParams(use_tc_tiling_on_sc=False),
)
def _pool(xp_hbm, tab_hbm, out_hbm, idx_v, buf0, buf1, out_v, sem0, sem1):
    wid = lax.axis_index("s") * NC + lax.axis_index("c")
    pltpu.sync_copy(xp_hbm.at[pl.ds(wid * NCHUNK, NCHUNK)], idx_v)
    pltpu.async_copy(tab_hbm.at[idx_v.at[0]], buf0, sem0)

    def _process(j, buf):
        acc = [
            [jnp.zeros((LN,), jnp.float32) for _ in range(D // LN)]
            for _ in range(RPC)
        ]
        for r in range(HIST):
            for half in range(RPC):
                for k in range(D // LN):
                    acc[half][k] = acc[half][k] + buf[half * HP + r, pl.ds(k * LN, LN)]
        for half in range(RPC):
            for k in range(D // LN):
                out_v[j * RPC + half, pl.ds(k * LN, LN)] = acc[half][k]

    def _tbody(t, carry):
        j0 = 2 * t
        pltpu.async_copy(tab_hbm.at[idx_v.at[j0 + 1]], buf1, sem1)
        pltpu.make_async_copy(tab_hbm.at[idx_v.at[j0]], buf0, sem0).wait()
        _process(j0, buf0)

        @pl.when(t < NCHUNK // 2 - 1)
        def _():
            pltpu.async_copy(tab_hbm.at[idx_v.at[j0 + 2]], buf0, sem0)

        pltpu.make_async_copy(tab_hbm.at[idx_v.at[j0 + 1]], buf1, sem1).wait()
        _process(j0 + 1, buf1)
        return carry

    lax.fori_loop(0, NCHUNK // 2, _tbody, 0)
    pltpu.sync_copy(out_v, out_hbm.at[pl.ds(wid * BPW, BPW)])


def _mlp_body(sum_ref, x_ref, w1_ref, b1_ref, w2_ref, b2_ref, out_ref):
    cnt = jnp.sum((x_ref[...] != 0).astype(jnp.float32), axis=1, keepdims=True)
    avg = sum_ref[...] / jnp.maximum(cnt, 1.0)
    h = jnp.dot(avg, w1_ref[...], preferred_element_type=jnp.float32)
    h = jnp.maximum(h + b1_ref[...][None, :], 0.0)
    out_ref[...] = (
        jnp.dot(h, w2_ref[...], preferred_element_type=jnp.float32)
        + b2_ref[...][None, :]
    )


_mlp = pl.pallas_call(
    _mlp_body,
    out_shape=jax.ShapeDtypeStruct((B, 1), jnp.float32),
)


def kernel(x, table, W1, b1, W2, b2):
    x = x.astype(jnp.int32)
    xp = jnp.pad(x, ((0, 0), (0, HP - HIST)))
    xp = xp.reshape(B // RPC, RPC * HP)
    sums = _pool(xp, table)
    return _mlp(sums, x, W1, b1, W2, b2)


# speedup vs baseline: 7.5034x; 7.5034x over previous
import functools

import jax
import jax.numpy as jnp
from jax import lax
from jax.experimental import pallas as pl
from jax.experimental.pallas import tpu as pltpu
from jax.experimental.pallas import tpu_sc as plsc

B = 4096
D = 64
HIST = 50
NC = 2
NS = 16
NW = NC * NS
BPW = B // NW
RPC = 2
NCHUNK = BPW // RPC
IDX_PER_CHUNK = RPC * HIST
NBUF = 4
LN = 16

_mesh = plsc.VectorSubcoreMesh(
    core_axis_name="c", subcore_axis_name="s", num_cores=NC, num_subcores=NS
)


@functools.partial(
    pl.kernel,
    out_type=jax.ShapeDtypeStruct((B, D), jnp.float32),
    mesh=_mesh,
    scratch_types=[
        pltpu.VMEM((NCHUNK, IDX_PER_CHUNK), jnp.int32),
        [pltpu.VMEM((IDX_PER_CHUNK, D), jnp.float32) for _ in range(NBUF)],
        pltpu.VMEM((BPW, D), jnp.float32),
        [pltpu.SemaphoreType.DMA for _ in range(NBUF)],
    ],
    compiler_params=pltpu.CompilerParams(use_tc_tiling_on_sc=False),
)
def _pool(xp_hbm, tab_hbm, out_hbm, idx_v, bufs, out_v, sems):
    wid = lax.axis_index("s") * NC + lax.axis_index("c")
    pltpu.sync_copy(xp_hbm.at[pl.ds(wid * NCHUNK, NCHUNK)], idx_v)
    for b in range(NBUF):
        pltpu.async_copy(tab_hbm.at[idx_v.at[b]], bufs[b], sems[b])

    def _process(j, buf):
        acc = [
            [jnp.zeros((LN,), jnp.float32) for _ in range(D // LN)]
            for _ in range(RPC)
        ]
        for r in range(HIST):
            for half in range(RPC):
                for k in range(D // LN):
                    acc[half][k] = acc[half][k] + buf[half * HIST + r, pl.ds(k * LN, LN)]
        for half in range(RPC):
            for k in range(D // LN):
                out_v[j * RPC + half, pl.ds(k * LN, LN)] = acc[half][k]

    def _tbody(t, carry):
        for b in range(NBUF):
            j = NBUF * t + b
            pltpu.make_async_copy(tab_hbm.at[idx_v.at[j]], bufs[b], sems[b]).wait()
            _process(j, bufs[b])

            @pl.when(j + NBUF < NCHUNK)
            def _():
                pltpu.async_copy(tab_hbm.at[idx_v.at[j + NBUF]], bufs[b], sems[b])

        return carry

    lax.fori_loop(0, NCHUNK // NBUF, _tbody, 0)
    pltpu.sync_copy(out_v, out_hbm.at[pl.ds(wid * BPW, BPW)])


def _mlp_body(sum_ref, x_ref, w1_ref, b1_ref, w2_ref, b2_ref, out_ref):
    cnt = jnp.sum((x_ref[...] != 0).astype(jnp.float32), axis=1, keepdims=True)
    avg = sum_ref[...] / jnp.maximum(cnt, 1.0)
    h = jnp.dot(avg, w1_ref[...], preferred_element_type=jnp.float32)
    h = jnp.maximum(h + b1_ref[...][None, :], 0.0)
    out_ref[...] = (
        jnp.dot(h, w2_ref[...], preferred_element_type=jnp.float32)
        + b2_ref[...][None, :]
    )


_mlp = pl.pallas_call(
    _mlp_body,
    out_shape=jax.ShapeDtypeStruct((B, 1), jnp.float32),
)


def kernel(x, table, W1, b1, W2, b2):
    x = x.astype(jnp.int32)
    xp = x.reshape(B // RPC, RPC * HIST)
    sums = _pool(xp, table)
    return _mlp(sums, x, W1, b1, W2, b2)
